# baseline (speedup 1.0000x reference)
import functools
import math

import jax
import jax.numpy as jnp
from jax import lax
from jax.experimental import pallas as pl
from jax.experimental.pallas import tpu as pltpu
from jax.experimental.pallas import tpu_sc as plsc

D_MODEL = 16
NUM_POS = 8192
N_FREQ = D_MODEL // 2


def _table_body(out_ref):
    g = lax.broadcasted_iota(jnp.int32, (512, 128), 0)
    l = lax.broadcasted_iota(jnp.int32, (512, 128), 1)
    pos = (g * 16 + l // 8).astype(jnp.float32)
    two_k = (l % 8 * 2).astype(jnp.float32)
    freq = jnp.exp(two_k * (-math.log(10000.0) / D_MODEL))
    ang = pos * freq
    sin_i = lax.bitcast_convert_type(
        jnp.sin(ang).astype(jnp.bfloat16), jnp.uint16).astype(jnp.int32)
    cos_i = lax.bitcast_convert_type(
        jnp.cos(ang).astype(jnp.bfloat16), jnp.uint16).astype(jnp.int32)
    out_ref[...] = (cos_i << 16) | sin_i


def _build_table():
    tab = pl.pallas_call(
        _table_body,
        out_shape=jax.ShapeDtypeStruct((512, 128), jnp.int32),
    )()
    return tab.reshape(NUM_POS * N_FREQ)


N_I = 16384
N_J = 200
NW = 32
IB = 4


def _make_lookup():
    mesh = plsc.VectorSubcoreMesh(core_axis_name="c", subcore_axis_name="s")

    @functools.partial(
        pl.kernel,
        mesh=mesh,
        out_type=jax.ShapeDtypeStruct((N_J, 2, N_I // 128, 8, 128), jnp.float32),
        scratch_types=[
            pltpu.VMEM((NUM_POS * N_FREQ,), jnp.int32),
            pltpu.VMEM((4, IB, 1, 128), jnp.int32),
            pltpu.VMEM((4, 2, IB, 8, 128), jnp.float32),
        ]
        + [pltpu.SemaphoreType.DMA] * 8,
        compiler_params=pltpu.CompilerParams(
            use_tc_tiling_on_sc=False, needs_layout_passes=False,
            disable_bounds_checks=True),
    )
    def lookup(tab_hbm, x4_hbm, out_hbm, tab_v, xv, outv, *sems):
        x_sems = sems[0:4]
        out_sems = sems[4:8]
        wid = lax.axis_index("s") * 2 + lax.axis_index("c")
        ib0 = wid * IB

        def x_copy(j, buf):
            return pltpu.make_async_copy(
                x4_hbm.at[j // 8, pl.ds(ib0, IB), pl.ds(j % 8, 1), :],
                xv.at[buf], x_sems[buf])

        def out_copy(j, buf):
            return pltpu.make_async_copy(
                outv.at[buf],
                out_hbm.at[j, :, pl.ds(ib0, IB), :, :], out_sems[buf])

        def compute(xb, ob):
            @plsc.parallel_loop(0, IB * 8, 1, unroll=4)
            def t_body(t):
                ib = t // 8
                o = (t % 8) * 16
                b8 = xv[xb, ib, 0, pl.ds(o, 16)] * 8
                for k in range(N_FREQ):
                    w = plsc.load_gather(tab_v, [b8 + k])
                    d0, d1 = 2 * k, 2 * k + 1
                    outv[ob, d0 // 8, ib, d0 % 8, pl.ds(o, 16)] = plsc.bitcast(
                        w << 16, jnp.float32)
                    outv[ob, d1 // 8, ib, d1 % 8, pl.ds(o, 16)] = plsc.bitcast(
                        w & jnp.int32(-65536), jnp.float32)

        def step(j, s, *, out_wait, x_start):
            x_copy(j, s).wait()
            if out_wait:
                out_copy(j - 4, s).wait()
            compute(s, s)
            out_copy(j, s).start()
            if x_start:
                x_copy(j + 4, s).start()

        for s in range(4):
            x_copy(s, s).start()
        pltpu.sync_copy(tab_hbm, tab_v)
        for s in range(4):
            step(s, s, out_wait=False, x_start=True)

        def round_body(q, carry):
            for s in range(4):
                step(4 * q + s, s, out_wait=True, x_start=True)
            return carry

        lax.fori_loop(1, N_J // 4 - 1, round_body, 0)
        for s in range(4):
            step(N_J - 4 + s, s, out_wait=True, x_start=False)
        for s in range(4):
            out_copy(N_J - 4 + s, s).wait()

    return lookup


_lookup_kernel = _make_lookup()


def kernel(x):
    table = _build_table()
    x4 = x.T.reshape(25, 8, 128, 128).transpose(0, 2, 1, 3)
    out5 = _lookup_kernel(table, x4)
    return out5.transpose(2, 4, 0, 1, 3).reshape(N_I, N_J, D_MODEL)


# speedup vs baseline: 1.1337x; 1.1337x over previous
import functools
import math

import jax
import jax.numpy as jnp
from jax import lax
from jax.experimental import pallas as pl
from jax.experimental.pallas import tpu as pltpu
from jax.experimental.pallas import tpu_sc as plsc

D_MODEL = 16
NUM_POS = 8192
N_FREQ = D_MODEL // 2


def _table_body(out_ref):
    g = lax.broadcasted_iota(jnp.int32, (512, 128), 0)
    l = lax.broadcasted_iota(jnp.int32, (512, 128), 1)
    pos = (g * 16 + l // 8).astype(jnp.float32)
    two_k = (l % 8 * 2).astype(jnp.float32)
    freq = jnp.exp(two_k * (-math.log(10000.0) / D_MODEL))
    ang = pos * freq
    sin_i = lax.bitcast_convert_type(
        jnp.sin(ang).astype(jnp.bfloat16), jnp.uint16).astype(jnp.int32)
    cos_i = lax.bitcast_convert_type(
        jnp.cos(ang).astype(jnp.bfloat16), jnp.uint16).astype(jnp.int32)
    out_ref[...] = (cos_i << 16) | sin_i


def _build_table():
    tab = pl.pallas_call(
        _table_body,
        out_shape=jax.ShapeDtypeStruct((512, 128), jnp.int32),
    )()
    return tab.reshape(NUM_POS * N_FREQ)


N_I = 16384
N_J = 200
NW = 32
IB = 4


def _make_lookup():
    mesh = plsc.VectorSubcoreMesh(core_axis_name="c", subcore_axis_name="s")

    @functools.partial(
        pl.kernel,
        mesh=mesh,
        out_type=jax.ShapeDtypeStruct((N_J, 2, N_I // 128, 8, 128), jnp.float32),
        scratch_types=[
            pltpu.VMEM((NUM_POS * N_FREQ,), jnp.int32),
            pltpu.VMEM((2, IB, 2, 128), jnp.int32),
            pltpu.VMEM((2, 2, 2, IB, 8, 128), jnp.float32),
        ]
        + [pltpu.SemaphoreType.DMA] * 5,
        compiler_params=pltpu.CompilerParams(
            use_tc_tiling_on_sc=False, needs_layout_passes=False,
            disable_bounds_checks=True),
    )
    def lookup(tab_hbm, x4_hbm, out_hbm, tab_v, xv, outv, *sems):
        x_sems = sems[0:2]
        out_sems = sems[2:4]
        tab_sem = sems[4]
        wid = lax.axis_index("s") * 2 + lax.axis_index("c")
        ib0 = wid * IB

        def x_copy(q, buf):
            return pltpu.make_async_copy(
                x4_hbm.at[q // 4, pl.ds(ib0, IB), pl.ds(q % 4 * 2, 2), :],
                xv.at[buf], x_sems[buf])

        def out_copy(q, buf):
            return pltpu.make_async_copy(
                outv.at[buf],
                out_hbm.at[pl.ds(2 * q, 2), :, pl.ds(ib0, IB), :, :],
                out_sems[buf])

        def compute(xb, ob):
            @plsc.parallel_loop(0, IB * 2 * 8, 1, unroll=2)
            def t_body(t):
                ib = t >> 4
                jl = (t >> 3) & 1
                o = (t & 7) * 16
                b8 = xv[xb, ib, jl, pl.ds(o, 16)] * 8
                for k in range(N_FREQ):
                    w = plsc.load_gather(tab_v, [b8 + k])
                    d0, d1 = 2 * k, 2 * k + 1
                    outv[ob, jl, d0 // 8, ib, d0 % 8, pl.ds(o, 16)] = (
                        plsc.bitcast(w << 16, jnp.float32))
                    outv[ob, jl, d1 // 8, ib, d1 % 8, pl.ds(o, 16)] = (
                        plsc.bitcast(w & jnp.int32(-65536), jnp.float32))

        def step(q, s, *, out_wait, x_start):
            x_copy(q, s).wait()
            if out_wait:
                out_copy(q - 2, s).wait()
            compute(s, s)
            out_copy(q, s).start()
            if x_start:
                x_copy(q + 2, s).start()

        x_copy(0, 0).start()
        x_copy(1, 1).start()
        tab_cp = pltpu.make_async_copy(tab_hbm, tab_v, tab_sem)
        tab_cp.start()
        tab_cp.wait()
        step(0, 0, out_wait=False, x_start=True)
        step(1, 1, out_wait=False, x_start=True)

        NQ = N_J // 2

        def round_body(q2, carry):
            step(2 * q2, 0, out_wait=True, x_start=True)
            step(2 * q2 + 1, 1, out_wait=True, x_start=True)
            return carry

        lax.fori_loop(1, NQ // 2 - 1, round_body, 0)
        step(NQ - 2, 0, out_wait=True, x_start=False)
        step(NQ - 1, 1, out_wait=True, x_start=False)
        out_copy(NQ - 2, 0).wait()
        out_copy(NQ - 1, 1).wait()

    return lookup


_lookup_kernel = _make_lookup()


def kernel(x):
    table = _build_table()
    x4 = x.T.reshape(25, 8, 128, 128).transpose(0, 2, 1, 3)
    out5 = _lookup_kernel(table, x4)
    return out5.transpose(2, 4, 0, 1, 3).reshape(N_I, N_J, D_MODEL)


# speedup vs baseline: 1.1630x; 1.0259x over previous
import functools
import math

import jax
import jax.numpy as jnp
from jax import lax
from jax.experimental import pallas as pl
from jax.experimental.pallas import tpu as pltpu
from jax.experimental.pallas import tpu_sc as plsc

D_MODEL = 16
NUM_POS = 8192
N_FREQ = D_MODEL // 2


def _table_body(out_ref):
    g = lax.broadcasted_iota(jnp.int32, (512, 128), 0)
    l = lax.broadcasted_iota(jnp.int32, (512, 128), 1)
    pos = (g * 16 + l // 8).astype(jnp.float32)
    two_k = (l % 8 * 2).astype(jnp.float32)
    freq = jnp.exp(two_k * (-math.log(10000.0) / D_MODEL))
    ang = pos * freq
    sin_i = lax.bitcast_convert_type(
        jnp.sin(ang).astype(jnp.bfloat16), jnp.uint16).astype(jnp.int32)
    cos_i = lax.bitcast_convert_type(
        jnp.cos(ang).astype(jnp.bfloat16), jnp.uint16).astype(jnp.int32)
    out_ref[...] = (cos_i << 16) | sin_i


def _build_table():
    tab = pl.pallas_call(
        _table_body,
        out_shape=jax.ShapeDtypeStruct((512, 128), jnp.int32),
    )()
    return tab.reshape(NUM_POS * N_FREQ)


N_I = 16384
N_J = 200
NW = 32
IB = 4


def _make_lookup():
    mesh = plsc.VectorSubcoreMesh(core_axis_name="c", subcore_axis_name="s")

    @functools.partial(
        pl.kernel,
        mesh=mesh,
        out_type=jax.ShapeDtypeStruct((N_J, 2, N_I // 128, 8, 128), jnp.float32),
        scratch_types=[
            pltpu.VMEM((NUM_POS * N_FREQ,), jnp.int32),
            pltpu.VMEM((2, IB, 2, 128), jnp.int32),
            pltpu.VMEM((2, 2, 2, IB, 8, 128), jnp.float32),
        ]
        + [pltpu.SemaphoreType.DMA] * 5,
        compiler_params=pltpu.CompilerParams(
            use_tc_tiling_on_sc=False, needs_layout_passes=False,
            disable_bounds_checks=True),
    )
    def lookup(tab_hbm, x4_hbm, out_hbm, tab_v, xv, outv, *sems):
        x_sems = sems[0:2]
        out_sems = sems[2:4]
        tab_sem = sems[4]
        wid = lax.axis_index("s") * 2 + lax.axis_index("c")
        ib0 = wid * IB

        def x_copy(q, buf):
            return pltpu.make_async_copy(
                x4_hbm.at[q // 4, pl.ds(ib0, IB), pl.ds(q % 4 * 2, 2), :],
                xv.at[buf], x_sems[buf])

        def out_copy(q, buf):
            return pltpu.make_async_copy(
                outv.at[buf],
                out_hbm.at[pl.ds(2 * q, 2), :, pl.ds(ib0, IB), :, :],
                out_sems[buf])

        def compute(xb, ob):
            @plsc.parallel_loop(0, IB * 2 * 8, 1, unroll=3)
            def t_body(t):
                ib = t >> 4
                jl = (t >> 3) & 1
                o = (t & 7) * 16
                b8 = xv[xb, ib, jl, pl.ds(o, 16)] * 8
                for k in range(N_FREQ):
                    w = plsc.load_gather(tab_v, [b8 + k])
                    d0, d1 = 2 * k, 2 * k + 1
                    outv[ob, jl, d0 // 8, ib, d0 % 8, pl.ds(o, 16)] = (
                        plsc.bitcast(w << 16, jnp.float32))
                    outv[ob, jl, d1 // 8, ib, d1 % 8, pl.ds(o, 16)] = (
                        plsc.bitcast(w & jnp.int32(-65536), jnp.float32))

        def step(q, s, *, out_wait, x_start):
            x_copy(q, s).wait()
            if out_wait:
                out_copy(q - 2, s).wait()
            compute(s, s)
            out_copy(q, s).start()
            if x_start:
                x_copy(q + 2, s).start()

        x_copy(0, 0).start()
        x_copy(1, 1).start()
        tab_cp = pltpu.make_async_copy(tab_hbm, tab_v, tab_sem)
        tab_cp.start()
        tab_cp.wait()
        step(0, 0, out_wait=False, x_start=True)
        step(1, 1, out_wait=False, x_start=True)

        NQ = N_J // 2

        def round_body(q2, carry):
            step(2 * q2, 0, out_wait=True, x_start=True)
            step(2 * q2 + 1, 1, out_wait=True, x_start=True)
            return carry

        lax.fori_loop(1, NQ // 2 - 1, round_body, 0)
        step(NQ - 2, 0, out_wait=True, x_start=False)
        step(NQ - 1, 1, out_wait=True, x_start=False)
        out_copy(NQ - 2, 0).wait()
        out_copy(NQ - 1, 1).wait()

    return lookup


_lookup_kernel = _make_lookup()


def kernel(x):
    table = _build_table()
    x4 = x.T.reshape(25, 8, 128, 128).transpose(0, 2, 1, 3)
    out5 = _lookup_kernel(table, x4)
    return out5.transpose(2, 4, 0, 1, 3).reshape(N_I, N_J, D_MODEL)
